# baseline (speedup 1.0000x reference)
import functools

import jax
import jax.numpy as jnp
from jax import lax
from jax.experimental import pallas as pl
from jax.experimental.pallas import tpu as pltpu
from jax.experimental.pallas import tpu_sc as plsc

_SEP = 2
_B, _S, _D = 16, 2048, 1024
_HALF = _D // 2
_L = 16
_KV = _HALF // _L
_CH = 64
_NCHUNK = _S // _CH
_NBUF = 2


def _sc_body(hid, ids, am, out1, out2,
             ids_v, am_v, acc, res1, res2, buf0, buf1, sem0, sem1):
    bufs = [buf0, buf1]
    sems = [sem0, sem1]
    h = lax.axis_index("c")
    b = lax.axis_index("s")

    pltpu.sync_copy(ids.at[b], ids_v)
    pltpu.sync_copy(am.at[b], am_v)
    dma = []
    for i in range(_NBUF):
        dma.append(pltpu.async_copy(
            hid.at[b, pl.ds(i * _CH, _CH), pl.ds(h * _HALF, _HALF)],
            bufs[i], sems[i]))

    zeros = jnp.zeros((_L,), jnp.float32)
    for k in range(3 * _KV):
        acc[pl.ds(k * _L, _L)] = zeros

    lane = lax.iota(jnp.int32, _L)

    def pass1(j, carry):
        m1v, vlv = carry
        v = ids_v[pl.ds(j * _L, _L)]
        a = am_v[pl.ds(j * _L, _L)]
        pos = lane + j * _L
        cand = jnp.where(v == _SEP, pos, _S)
        return jnp.minimum(m1v, cand), vlv + a

    m1v, vlv = lax.fori_loop(
        0, _S // _L, pass1,
        (jnp.full((_L,), _S, jnp.int32), jnp.zeros((_L,), jnp.int32)))
    sep1 = m1v[0]
    valid_len = vlv[0]
    for i in range(1, _L):
        sep1 = jnp.minimum(sep1, m1v[i])
        valid_len = valid_len + vlv[i]

    def pass2(j, m2v):
        v = ids_v[pl.ds(j * _L, _L)]
        pos = lane + j * _L
        cand = jnp.where((v == _SEP) & (pos > sep1), pos, _S)
        return jnp.minimum(m2v, cand)

    m2v = lax.fori_loop(0, _S // _L, pass2, jnp.full((_L,), _S, jnp.int32))
    sep2 = m2v[0]
    for i in range(1, _L):
        sep2 = jnp.minimum(sep2, m2v[i])

    end_pos = jnp.minimum(valid_len - 1, _S)
    has_two = sep2 < _S
    fb_big = valid_len > 2
    fb_lo = jnp.where(fb_big, 1, 0)
    fb_hi = jnp.where(fb_big, valid_len - 1, 1)
    lo1 = jnp.where(has_two, 1, fb_lo)
    hi1 = jnp.where(has_two, sep1, fb_hi)
    lo2 = jnp.where(has_two, sep2 + 1, 0)
    hi2 = jnp.where(has_two, end_pos, 0)

    def group_body(g, carry):
        for i in range(_NBUF):
            c = g * _NBUF + i
            base = c * _CH
            pltpu.make_async_copy(
                hid.at[b, pl.ds(0, _CH), pl.ds(h * _HALF, _HALF)],
                bufs[i], sems[i]).wait()

            def row_body(j, cc, bufref=bufs[i], base=base):
                p = base + j
                in1 = (p >= lo1) & (p < hi1)
                in2 = (p >= lo2) & (p < hi2)
                t = jnp.where(in1, 0, jnp.where(in2, _HALF, 2 * _HALF))
                for k in range(_KV):
                    x = bufref[j, pl.ds(k * _L, _L)]
                    plsc.addupdate(acc.at[pl.ds(t + k * _L, _L)], x)
                return cc

            lax.fori_loop(0, _CH, row_body, 0)

            nxt = c + _NBUF

            @pl.when(nxt < _NCHUNK)
            def _():
                pltpu.async_copy(
                    hid.at[b, pl.ds(nxt * _CH, _CH), pl.ds(h * _HALF, _HALF)],
                    bufs[i], sems[i])
        return carry

    lax.fori_loop(0, _NCHUNK // _NBUF, group_body, 0)

    def _recip(x):
        xi = lax.bitcast_convert_type(x, jnp.int32)
        y = lax.bitcast_convert_type(jnp.int32(0x7EF311C3) - xi, jnp.float32)
        for _ in range(3):
            y = y * (2.0 - x * y)
        return y

    cnt1 = jnp.maximum(hi1 - lo1, 0)
    cnt2 = jnp.maximum(hi2 - lo2, 0)
    inv1 = _recip(jnp.maximum(cnt1, 1).astype(jnp.float32))
    inv2 = _recip(jnp.maximum(cnt2, 1).astype(jnp.float32))
    src2 = jnp.where(has_two, _HALF, 0)
    inv2 = jnp.where(has_two, inv2, inv1)
    inv1v = jnp.full((_L,), inv1, jnp.float32)
    inv2v = jnp.full((_L,), inv2, jnp.float32)
    for k in range(_KV):
        res1[pl.ds(k * _L, _L)] = acc[pl.ds(k * _L, _L)] * inv1v
        res2[pl.ds(k * _L, _L)] = acc[pl.ds(src2 + k * _L, _L)] * inv2v
    pltpu.sync_copy(res1, out1.at[b, pl.ds(h * _HALF, _HALF)])
    pltpu.sync_copy(res2, out2.at[b, pl.ds(h * _HALF, _HALF)])


@jax.jit
def kernel(hidden_states, input_ids, attention_mask):
    f = pl.kernel(
        _sc_body,
        out_type=(jax.ShapeDtypeStruct((_B, _D), jnp.float32),
                  jax.ShapeDtypeStruct((_B, _D), jnp.float32)),
        mesh=plsc.VectorSubcoreMesh(core_axis_name="c", subcore_axis_name="s",
                                    num_cores=2, num_subcores=16),
        scratch_types=[
            pltpu.VMEM((_S,), jnp.int32),
            pltpu.VMEM((_S,), jnp.int32),
            pltpu.VMEM((3 * _HALF,), jnp.float32),
            pltpu.VMEM((_HALF,), jnp.float32),
            pltpu.VMEM((_HALF,), jnp.float32),
            pltpu.VMEM((_CH, _HALF), jnp.float32),
            pltpu.VMEM((_CH, _HALF), jnp.float32),
            pltpu.SemaphoreType.DMA,
            pltpu.SemaphoreType.DMA,
        ],
    )
    return f(hidden_states,
             input_ids.astype(jnp.int32),
             attention_mask.astype(jnp.int32))


# speedup vs baseline: 3.1686x; 3.1686x over previous
import functools

import jax
import jax.numpy as jnp
from jax import lax
from jax.experimental import pallas as pl
from jax.experimental.pallas import tpu as pltpu
from jax.experimental.pallas import tpu_sc as plsc

_SEP = 2
_B, _S, _D = 16, 2048, 1024
_HALF = _D // 2
_L = 16
_KV = _HALF // _L
_CH = 64
_NCHUNK = _S // _CH
_NBUF = 2


def _sc_body(hid, ids, am, out1, out2,
             ids_v, am_v, acc, res1, res2, buf0, buf1, sem0, sem1):
    bufs = [buf0, buf1]
    sems = [sem0, sem1]
    h = lax.axis_index("c")
    b = lax.axis_index("s")

    pltpu.sync_copy(ids.at[b], ids_v)
    pltpu.sync_copy(am.at[b], am_v)
    dma = []
    for i in range(_NBUF):
        dma.append(pltpu.async_copy(
            hid.at[b, pl.ds(i * _CH, _CH), pl.ds(h * _HALF, _HALF)],
            bufs[i], sems[i]))

    zeros = jnp.zeros((_L,), jnp.float32)
    for k in range(2 * _KV):
        acc[pl.ds(k * _L, _L)] = zeros

    lane = lax.iota(jnp.int32, _L)

    def pass1(j, carry):
        m1v, vlv = carry
        v = ids_v[pl.ds(j * _L, _L)]
        a = am_v[pl.ds(j * _L, _L)]
        pos = lane + j * _L
        cand = jnp.where(v == _SEP, pos, _S)
        return jnp.minimum(m1v, cand), vlv + a

    m1v, vlv = lax.fori_loop(
        0, _S // _L, pass1,
        (jnp.full((_L,), _S, jnp.int32), jnp.zeros((_L,), jnp.int32)))
    sep1 = m1v[0]
    valid_len = vlv[0]
    for i in range(1, _L):
        sep1 = jnp.minimum(sep1, m1v[i])
        valid_len = valid_len + vlv[i]

    def pass2(j, m2v):
        v = ids_v[pl.ds(j * _L, _L)]
        pos = lane + j * _L
        cand = jnp.where((v == _SEP) & (pos > sep1), pos, _S)
        return jnp.minimum(m2v, cand)

    m2v = lax.fori_loop(0, _S // _L, pass2, jnp.full((_L,), _S, jnp.int32))
    sep2 = m2v[0]
    for i in range(1, _L):
        sep2 = jnp.minimum(sep2, m2v[i])

    end_pos = jnp.minimum(valid_len - 1, _S)
    has_two = sep2 < _S
    fb_big = valid_len > 2
    fb_lo = jnp.where(fb_big, 1, 0)
    fb_hi = jnp.where(fb_big, valid_len - 1, 1)
    lo1 = jnp.where(has_two, 1, fb_lo)
    hi1 = jnp.where(has_two, sep1, fb_hi)
    lo2 = jnp.where(has_two, sep2 + 1, 0)
    hi2 = jnp.where(has_two, end_pos, 0)

    zero_accs = tuple(jnp.zeros((_L,), jnp.float32) for _ in range(_KV))

    def seg_accumulate(bufref, a, bnd, acc_off):
        def row_body(j, accs):
            return tuple(accs[k] + bufref[j, pl.ds(k * _L, _L)]
                         for k in range(_KV))

        accs = lax.fori_loop(a, bnd, row_body, zero_accs)
        for k in range(_KV):
            plsc.addupdate(acc.at[pl.ds(acc_off + k * _L, _L)], accs[k])

    def group_body(g, carry):
        for i in range(_NBUF):
            c = g * _NBUF + i
            base = c * _CH
            pltpu.make_async_copy(
                hid.at[b, pl.ds(0, _CH), pl.ds(h * _HALF, _HALF)],
                bufs[i], sems[i]).wait()

            a1 = jnp.clip(lo1 - base, 0, _CH)
            b1 = jnp.clip(hi1 - base, 0, _CH)
            a2 = jnp.clip(lo2 - base, 0, _CH)
            b2 = jnp.clip(hi2 - base, 0, _CH)
            seg_accumulate(bufs[i], a1, b1, 0)
            seg_accumulate(bufs[i], a2, b2, _HALF)

            nxt = c + _NBUF

            @pl.when(nxt < _NCHUNK)
            def _():
                pltpu.async_copy(
                    hid.at[b, pl.ds(nxt * _CH, _CH), pl.ds(h * _HALF, _HALF)],
                    bufs[i], sems[i])
        return carry

    lax.fori_loop(0, _NCHUNK // _NBUF, group_body, 0)

    def _recip(x):
        xi = lax.bitcast_convert_type(x, jnp.int32)
        y = lax.bitcast_convert_type(jnp.int32(0x7EF311C3) - xi, jnp.float32)
        for _ in range(3):
            y = y * (2.0 - x * y)
        return y

    cnt1 = jnp.maximum(hi1 - lo1, 0)
    cnt2 = jnp.maximum(hi2 - lo2, 0)
    inv1 = _recip(jnp.maximum(cnt1, 1).astype(jnp.float32))
    inv2 = _recip(jnp.maximum(cnt2, 1).astype(jnp.float32))
    src2 = jnp.where(has_two, _HALF, 0)
    inv2 = jnp.where(has_two, inv2, inv1)
    inv1v = jnp.full((_L,), inv1, jnp.float32)
    inv2v = jnp.full((_L,), inv2, jnp.float32)
    for k in range(_KV):
        res1[pl.ds(k * _L, _L)] = acc[pl.ds(k * _L, _L)] * inv1v
        res2[pl.ds(k * _L, _L)] = acc[pl.ds(src2 + k * _L, _L)] * inv2v
    pltpu.sync_copy(res1, out1.at[b, pl.ds(h * _HALF, _HALF)])
    pltpu.sync_copy(res2, out2.at[b, pl.ds(h * _HALF, _HALF)])


@jax.jit
def kernel(hidden_states, input_ids, attention_mask):
    f = pl.kernel(
        _sc_body,
        out_type=(jax.ShapeDtypeStruct((_B, _D), jnp.float32),
                  jax.ShapeDtypeStruct((_B, _D), jnp.float32)),
        mesh=plsc.VectorSubcoreMesh(core_axis_name="c", subcore_axis_name="s",
                                    num_cores=2, num_subcores=16),
        scratch_types=[
            pltpu.VMEM((_S,), jnp.int32),
            pltpu.VMEM((_S,), jnp.int32),
            pltpu.VMEM((2 * _HALF,), jnp.float32),
            pltpu.VMEM((_HALF,), jnp.float32),
            pltpu.VMEM((_HALF,), jnp.float32),
            pltpu.VMEM((_CH, _HALF), jnp.float32),
            pltpu.VMEM((_CH, _HALF), jnp.float32),
            pltpu.SemaphoreType.DMA,
            pltpu.SemaphoreType.DMA,
        ],
    )
    return f(hidden_states,
             input_ids.astype(jnp.int32),
             attention_mask.astype(jnp.int32))
